# baseline (speedup 1.0000x reference)
import functools

import jax
import jax.numpy as jnp
from jax import lax
from jax.experimental import pallas as pl
from jax.experimental.pallas import tpu as pltpu
from jax.experimental.pallas import tpu_sc as plsc

N = 10000
D = 32
F_IN = 128
HID = 256
NCLS = 16

NP = 10240
BLK = 512
NB = NP // BLK
CHUNK = 128
NSC = 2
NTILE = 16
NW = NSC * NTILE


NBUF = 4


def _sc_gather(table, idx3, feat, dtype):
    _, n_chunks, chunk = idx3.shape
    per_w = n_chunks * chunk
    rows_total = NW * per_w
    n_iters = n_chunks // NBUF
    mesh = plsc.VectorSubcoreMesh(core_axis_name="c", subcore_axis_name="s")

    @functools.partial(
        pl.kernel,
        mesh=mesh,
        out_type=jax.ShapeDtypeStruct((rows_total, feat), dtype),
        scratch_types=[
            pltpu.VMEM((n_chunks, chunk), jnp.int32),
        ]
        + [pltpu.VMEM((chunk, feat), dtype) for _ in range(NBUF)]
        + [pltpu.SemaphoreType.DMA for _ in range(2 * NBUF)],
    )
    def gk(table_hbm, idx_hbm, out_hbm, idx_v, *rest):
        bufs = rest[:NBUF]
        gsems = rest[NBUF:2 * NBUF]
        osems = rest[2 * NBUF:]
        wid = lax.axis_index("s") * NSC + lax.axis_index("c")
        base = wid * per_w
        pltpu.sync_copy(idx_hbm.at[wid], idx_v)
        for b in range(NBUF):
            pltpu.async_copy(table_hbm.at[idx_v.at[b]], bufs[b], gsems[b])

        def body(k, carry):
            for b in range(NBUF):
                j = k * NBUF + b
                pltpu.make_async_copy(
                    table_hbm.at[idx_v.at[j]], bufs[b], gsems[b]).wait()
                out_slice = out_hbm.at[pl.ds(base + j * chunk, chunk)]
                pltpu.async_copy(bufs[b], out_slice, osems[b])

                @pl.when(k < n_iters - 1)
                def _():
                    pltpu.make_async_copy(bufs[b], out_slice, osems[b]).wait()
                    pltpu.async_copy(
                        table_hbm.at[idx_v.at[j + NBUF]], bufs[b], gsems[b])
            return carry

        lax.fori_loop(0, n_iters, body, 0)
        for b in range(NBUF):
            j = (n_iters - 1) * NBUF + b
            out_slice = out_hbm.at[pl.ds(base + j * chunk, chunk)]
            pltpu.make_async_copy(bufs[b], out_slice, osems[b]).wait()

    return gk(table, idx3)


def _conv1(m, xp, W_ihT, W_hhT, bias, W_self1, W_neigh1, b1, W_pool, b_pool):
    def body(m_ref, x_ref, wih_ref, whh_ref, b_ref, ws_ref, wn_ref, b1_ref,
             wp_ref, bp_ref, out_ref, q_ref, h_s, c_s):
        d = pl.program_id(1)

        @pl.when(d == 0)
        def _():
            h_s[...] = jnp.zeros_like(h_s)
            c_s[...] = jnp.zeros_like(c_s)

        xt = m_ref[0].astype(jnp.bfloat16)
        hb = h_s[...]
        pre_if = (jnp.dot(xt, wih_ref[:, 0:2 * F_IN],
                          preferred_element_type=jnp.float32)
                  + jnp.dot(hb, whh_ref[:, 0:2 * F_IN],
                            preferred_element_type=jnp.float32)
                  + b_ref[:, 0:2 * F_IN])
        pre_go = (jnp.dot(xt, wih_ref[:, 2 * F_IN:4 * F_IN],
                          preferred_element_type=jnp.float32)
                  + jnp.dot(hb, whh_ref[:, 2 * F_IN:4 * F_IN],
                            preferred_element_type=jnp.float32)
                  + b_ref[:, 2 * F_IN:4 * F_IN])
        def sigm(v):
            return 0.5 * jnp.tanh(v) + 0.5

        gi = sigm(pre_if[:, 0:F_IN])
        gf = sigm(pre_if[:, F_IN:2 * F_IN])
        gg = jnp.tanh(pre_go[:, 0:F_IN])
        go = sigm(pre_go[:, F_IN:2 * F_IN])
        c = gf * c_s[...] + gi * gg
        h = go * jnp.tanh(c)
        c_s[...] = c
        h_s[...] = h.astype(jnp.bfloat16)

        @pl.when(d == D - 1)
        def _():
            hh = jax.nn.relu(
                jnp.dot(x_ref[...], ws_ref[...], preferred_element_type=jnp.float32)
                + jnp.dot(h.astype(jnp.bfloat16), wn_ref[...],
                          preferred_element_type=jnp.float32)
                + b1_ref[...])
            out_ref[...] = hh
            q_ref[...] = jax.nn.relu(
                jnp.dot(hh.astype(jnp.bfloat16), wp_ref[...],
                        preferred_element_type=jnp.float32)
                + bp_ref[...])

    cn = m.shape[1]
    return pl.pallas_call(
        body,
        grid=(cn // BLK, D),
        in_specs=[
            pl.BlockSpec((1, BLK, F_IN), lambda i, d: (d, i, 0)),
            pl.BlockSpec((BLK, F_IN), lambda i, d: (i, 0)),
            pl.BlockSpec((F_IN, 4 * F_IN), lambda i, d: (0, 0)),
            pl.BlockSpec((F_IN, 4 * F_IN), lambda i, d: (0, 0)),
            pl.BlockSpec((1, 4 * F_IN), lambda i, d: (0, 0)),
            pl.BlockSpec((F_IN, HID), lambda i, d: (0, 0)),
            pl.BlockSpec((F_IN, HID), lambda i, d: (0, 0)),
            pl.BlockSpec((1, HID), lambda i, d: (0, 0)),
            pl.BlockSpec((HID, HID), lambda i, d: (0, 0)),
            pl.BlockSpec((1, HID), lambda i, d: (0, 0)),
        ],
        out_specs=[
            pl.BlockSpec((BLK, HID), lambda i, d: (i, 0)),
            pl.BlockSpec((BLK, HID), lambda i, d: (i, 0)),
        ],
        out_shape=[
            jax.ShapeDtypeStruct((cn, HID), jnp.float32),
            jax.ShapeDtypeStruct((cn, HID), jnp.float32),
        ],
        scratch_shapes=[
            pltpu.VMEM((BLK, F_IN), jnp.bfloat16),
            pltpu.VMEM((BLK, F_IN), jnp.float32),
        ],
    )(m, xp, W_ihT, W_hhT, bias, W_self1, W_neigh1, b1, W_pool, b_pool)


def _conv2(m2, h, W_self2, W_neigh2, b2):
    def body(m_ref, h_ref, ws_ref, wn_ref, b2_ref, out_ref, mx_s):
        d = pl.program_id(1)
        t = m_ref[0]
        prev = jnp.where(d == 0, jnp.zeros_like(t), mx_s[...])
        mx = jnp.maximum(t, prev)
        mx_s[...] = mx

        @pl.when(d == D - 1)
        def _():
            out_ref[...] = (
                jnp.dot(h_ref[...].astype(jnp.bfloat16), ws_ref[...],
                        preferred_element_type=jnp.float32)
                + jnp.dot(mx.astype(jnp.bfloat16), wn_ref[...],
                          preferred_element_type=jnp.float32)
                + b2_ref[...])

    cn = m2.shape[1]
    return pl.pallas_call(
        body,
        grid=(cn // BLK, D),
        in_specs=[
            pl.BlockSpec((1, BLK, HID), lambda i, d: (d, i, 0)),
            pl.BlockSpec((BLK, HID), lambda i, d: (i, 0)),
            pl.BlockSpec((HID, NCLS), lambda i, d: (0, 0)),
            pl.BlockSpec((HID, NCLS), lambda i, d: (0, 0)),
            pl.BlockSpec((1, NCLS), lambda i, d: (0, 0)),
        ],
        out_specs=pl.BlockSpec((BLK, NCLS), lambda i, d: (i, 0)),
        out_shape=jax.ShapeDtypeStruct((cn, NCLS), jnp.float32),
        scratch_shapes=[pltpu.VMEM((BLK, HID), jnp.float32)],
    )(m2, h, W_self2, W_neigh2, b2)


CHUNKS_A = (1024, 1536, 3584, 4096)
CHUNKS_B = (4096, 3584, 1536, 1024)


def kernel(x, nbr, W_ih, W_hh, b_ih, b_hh, W_self1, W_neigh1, b1,
           W_pool, b_pool, W_self2, W_neigh2, b2):
    bf = jnp.bfloat16
    nbr = nbr.astype(jnp.int32)
    xp = jnp.pad(x.astype(bf), ((0, NP - N), (0, 0)))
    idxt = jnp.pad(nbr.T, ((0, 0), (0, NP - N)))
    c1 = 16384 // F_IN
    c2 = 16384 // HID

    def chunk_idx(off, cn):
        return idxt[:, off:off + cn].reshape(-1)

    gate_scale = jnp.concatenate([
        jnp.full((2 * F_IN,), 0.5, jnp.float32),
        jnp.ones((F_IN,), jnp.float32),
        jnp.full((F_IN,), 0.5, jnp.float32),
    ])
    bias = ((b_ih + b_hh) * gate_scale).reshape(1, 4 * F_IN)
    wih = (W_ih.T * gate_scale[None, :]).astype(bf)
    whh = (W_hh.T * gate_scale[None, :]).astype(bf)
    ws1, wn1 = W_self1.astype(bf), W_neigh1.astype(bf)
    wp = W_pool.astype(bf)
    ws2, wn2 = W_self2.astype(bf), W_neigh2.astype(bf)

    hs, qs = [], []
    off = 0
    for cn in CHUNKS_A:
        per_w = (D * cn) // NW
        mk = _sc_gather(x, chunk_idx(off, cn).reshape(NW, per_w // c1, c1),
                        F_IN, jnp.float32).reshape(D, cn, F_IN)
        hk, qk = _conv1(mk, xp[off:off + cn], wih, whh, bias,
                        ws1, wn1, b1.reshape(1, HID), wp,
                        b_pool.reshape(1, HID))
        hs.append(hk)
        qs.append(qk)
        off += cn
    q = jnp.concatenate(qs, axis=0)
    h = jnp.concatenate(hs, axis=0)

    outs = []
    off = 0
    for cn in CHUNKS_B:
        per_w = (D * cn) // NW
        m2k = _sc_gather(q, chunk_idx(off, cn).reshape(NW, per_w // c2, c2),
                         HID, jnp.float32).reshape(D, cn, HID)
        outs.append(_conv2(m2k, h[off:off + cn], ws2, wn2,
                           b2.reshape(1, NCLS)))
        off += cn
    return jnp.concatenate(outs, axis=0)[:N]


# speedup vs baseline: 1.0668x; 1.0668x over previous
import functools

import jax
import jax.numpy as jnp
from jax import lax
from jax.experimental import pallas as pl
from jax.experimental.pallas import tpu as pltpu
from jax.experimental.pallas import tpu_sc as plsc

N = 10000
D = 32
F_IN = 128
HID = 256
NCLS = 16

NP = 10240
BLK = 512
NB = NP // BLK
CHUNK = 128
NSC = 2
NTILE = 16
NW = NSC * NTILE


NBUF = 4


def _sc_gather(table, idx3, feat, dtype):
    _, n_chunks, chunk = idx3.shape
    per_w = n_chunks * chunk
    rows_total = NW * per_w
    n_iters = n_chunks // NBUF
    mesh = plsc.VectorSubcoreMesh(core_axis_name="c", subcore_axis_name="s")

    @functools.partial(
        pl.kernel,
        mesh=mesh,
        out_type=jax.ShapeDtypeStruct((rows_total, feat), dtype),
        scratch_types=[
            pltpu.VMEM((n_chunks, chunk), jnp.int32),
        ]
        + [pltpu.VMEM((chunk, feat), dtype) for _ in range(NBUF)]
        + [pltpu.SemaphoreType.DMA for _ in range(2 * NBUF)],
    )
    def gk(table_hbm, idx_hbm, out_hbm, idx_v, *rest):
        bufs = rest[:NBUF]
        gsems = rest[NBUF:2 * NBUF]
        osems = rest[2 * NBUF:]
        wid = lax.axis_index("s") * NSC + lax.axis_index("c")
        base = wid * per_w
        pltpu.sync_copy(idx_hbm.at[wid], idx_v)
        for b in range(NBUF):
            pltpu.async_copy(table_hbm.at[idx_v.at[b]], bufs[b], gsems[b])

        def body(k, carry):
            for b in range(NBUF):
                j = k * NBUF + b
                pltpu.make_async_copy(
                    table_hbm.at[idx_v.at[j]], bufs[b], gsems[b]).wait()
                out_slice = out_hbm.at[pl.ds(base + j * chunk, chunk)]
                pltpu.async_copy(bufs[b], out_slice, osems[b])

                @pl.when(k < n_iters - 1)
                def _():
                    pltpu.make_async_copy(bufs[b], out_slice, osems[b]).wait()
                    pltpu.async_copy(
                        table_hbm.at[idx_v.at[j + NBUF]], bufs[b], gsems[b])
            return carry

        lax.fori_loop(0, n_iters, body, 0)
        for b in range(NBUF):
            j = (n_iters - 1) * NBUF + b
            out_slice = out_hbm.at[pl.ds(base + j * chunk, chunk)]
            pltpu.make_async_copy(bufs[b], out_slice, osems[b]).wait()

    return gk(table, idx3)


def _conv1(m, xp, W_ihT, W_hhT, bias, W_self1, W_neigh1, b1, W_pool, b_pool):
    def body(m_ref, x_ref, wih_ref, whh_ref, b_ref, ws_ref, wn_ref, b1_ref,
             wp_ref, bp_ref, out_ref, q_ref, h_s, c_s):
        d = pl.program_id(1)

        @pl.when(d == 0)
        def _():
            h_s[...] = jnp.zeros_like(h_s)
            c_s[...] = jnp.zeros_like(c_s)

        xt = m_ref[0].astype(jnp.bfloat16)
        hb = h_s[...]
        pre_if = (jnp.dot(xt, wih_ref[:, 0:2 * F_IN],
                          preferred_element_type=jnp.float32)
                  + jnp.dot(hb, whh_ref[:, 0:2 * F_IN],
                            preferred_element_type=jnp.float32)
                  + b_ref[:, 0:2 * F_IN])
        pre_go = (jnp.dot(xt, wih_ref[:, 2 * F_IN:4 * F_IN],
                          preferred_element_type=jnp.float32)
                  + jnp.dot(hb, whh_ref[:, 2 * F_IN:4 * F_IN],
                            preferred_element_type=jnp.float32)
                  + b_ref[:, 2 * F_IN:4 * F_IN])
        def sigm(v):
            return 0.5 * jnp.tanh(v) + 0.5

        gi = sigm(pre_if[:, 0:F_IN])
        gf = sigm(pre_if[:, F_IN:2 * F_IN])
        gg = jnp.tanh(pre_go[:, 0:F_IN])
        go = sigm(pre_go[:, F_IN:2 * F_IN])
        c = gf * c_s[...] + gi * gg
        h = go * jnp.tanh(c)
        c_s[...] = c
        h_s[...] = h.astype(jnp.bfloat16)

        @pl.when(d == D - 1)
        def _():
            hh = jax.nn.relu(
                jnp.dot(x_ref[...], ws_ref[...], preferred_element_type=jnp.float32)
                + jnp.dot(h.astype(jnp.bfloat16), wn_ref[...],
                          preferred_element_type=jnp.float32)
                + b1_ref[...])
            out_ref[...] = hh
            q_ref[...] = jax.nn.relu(
                jnp.dot(hh.astype(jnp.bfloat16), wp_ref[...],
                        preferred_element_type=jnp.float32)
                + bp_ref[...])

    cn = m.shape[1]
    return pl.pallas_call(
        body,
        grid=(cn // BLK, D),
        in_specs=[
            pl.BlockSpec((1, BLK, F_IN), lambda i, d: (d, i, 0)),
            pl.BlockSpec((BLK, F_IN), lambda i, d: (i, 0)),
            pl.BlockSpec((F_IN, 4 * F_IN), lambda i, d: (0, 0)),
            pl.BlockSpec((F_IN, 4 * F_IN), lambda i, d: (0, 0)),
            pl.BlockSpec((1, 4 * F_IN), lambda i, d: (0, 0)),
            pl.BlockSpec((F_IN, HID), lambda i, d: (0, 0)),
            pl.BlockSpec((F_IN, HID), lambda i, d: (0, 0)),
            pl.BlockSpec((1, HID), lambda i, d: (0, 0)),
            pl.BlockSpec((HID, HID), lambda i, d: (0, 0)),
            pl.BlockSpec((1, HID), lambda i, d: (0, 0)),
        ],
        out_specs=[
            pl.BlockSpec((BLK, HID), lambda i, d: (i, 0)),
            pl.BlockSpec((BLK, HID), lambda i, d: (i, 0)),
        ],
        out_shape=[
            jax.ShapeDtypeStruct((cn, HID), jnp.float32),
            jax.ShapeDtypeStruct((cn, HID), jnp.float32),
        ],
        scratch_shapes=[
            pltpu.VMEM((BLK, F_IN), jnp.bfloat16),
            pltpu.VMEM((BLK, F_IN), jnp.float32),
        ],
    )(m, xp, W_ihT, W_hhT, bias, W_self1, W_neigh1, b1, W_pool, b_pool)


def _conv2(m2, h, W_self2, W_neigh2, b2):
    def body(m_ref, h_ref, ws_ref, wn_ref, b2_ref, out_ref, mx_s):
        d = pl.program_id(1)
        t = m_ref[0]
        prev = jnp.where(d == 0, jnp.zeros_like(t), mx_s[...])
        mx = jnp.maximum(t, prev)
        mx_s[...] = mx

        @pl.when(d == D - 1)
        def _():
            out_ref[...] = (
                jnp.dot(h_ref[...].astype(jnp.bfloat16), ws_ref[...],
                        preferred_element_type=jnp.float32)
                + jnp.dot(mx.astype(jnp.bfloat16), wn_ref[...],
                          preferred_element_type=jnp.float32)
                + b2_ref[...])

    cn = m2.shape[1]
    return pl.pallas_call(
        body,
        grid=(cn // BLK, D),
        in_specs=[
            pl.BlockSpec((1, BLK, HID), lambda i, d: (d, i, 0)),
            pl.BlockSpec((BLK, HID), lambda i, d: (i, 0)),
            pl.BlockSpec((HID, NCLS), lambda i, d: (0, 0)),
            pl.BlockSpec((HID, NCLS), lambda i, d: (0, 0)),
            pl.BlockSpec((1, NCLS), lambda i, d: (0, 0)),
        ],
        out_specs=pl.BlockSpec((BLK, NCLS), lambda i, d: (i, 0)),
        out_shape=jax.ShapeDtypeStruct((cn, NCLS), jnp.float32),
        scratch_shapes=[pltpu.VMEM((BLK, HID), jnp.float32)],
    )(m2, h, W_self2, W_neigh2, b2)


CHUNKS_A = (2560, 2560, 2560, 2560)
CHUNKS_B = (2560, 2560, 2560, 2560)


def kernel(x, nbr, W_ih, W_hh, b_ih, b_hh, W_self1, W_neigh1, b1,
           W_pool, b_pool, W_self2, W_neigh2, b2):
    bf = jnp.bfloat16
    nbr = nbr.astype(jnp.int32)
    xp = jnp.pad(x.astype(bf), ((0, NP - N), (0, 0)))
    idxt = jnp.pad(nbr.T, ((0, 0), (0, NP - N)))
    c1 = 16384 // F_IN
    c2 = 16384 // HID

    def chunk_idx(off, cn):
        return idxt[:, off:off + cn].reshape(-1)

    gate_scale = jnp.concatenate([
        jnp.full((2 * F_IN,), 0.5, jnp.float32),
        jnp.ones((F_IN,), jnp.float32),
        jnp.full((F_IN,), 0.5, jnp.float32),
    ])
    bias = ((b_ih + b_hh) * gate_scale).reshape(1, 4 * F_IN)
    wih = (W_ih.T * gate_scale[None, :]).astype(bf)
    whh = (W_hh.T * gate_scale[None, :]).astype(bf)
    ws1, wn1 = W_self1.astype(bf), W_neigh1.astype(bf)
    wp = W_pool.astype(bf)
    ws2, wn2 = W_self2.astype(bf), W_neigh2.astype(bf)

    hs, qs = [], []
    off = 0
    for cn in CHUNKS_A:
        per_w = (D * cn) // NW
        mk = _sc_gather(x, chunk_idx(off, cn).reshape(NW, per_w // c1, c1),
                        F_IN, jnp.float32).reshape(D, cn, F_IN)
        hk, qk = _conv1(mk, xp[off:off + cn], wih, whh, bias,
                        ws1, wn1, b1.reshape(1, HID), wp,
                        b_pool.reshape(1, HID))
        hs.append(hk)
        qs.append(qk)
        off += cn
    q = jnp.concatenate(qs, axis=0)
    h = jnp.concatenate(hs, axis=0)

    outs = []
    off = 0
    for cn in CHUNKS_B:
        per_w = (D * cn) // NW
        m2k = _sc_gather(q, chunk_idx(off, cn).reshape(NW, per_w // c2, c2),
                         HID, jnp.float32).reshape(D, cn, HID)
        outs.append(_conv2(m2k, h[off:off + cn], ws2, wn2,
                           b2.reshape(1, NCLS)))
        off += cn
    return jnp.concatenate(outs, axis=0)[:N]


# speedup vs baseline: 1.0689x; 1.0020x over previous
import functools

import jax
import jax.numpy as jnp
from jax import lax
from jax.experimental import pallas as pl
from jax.experimental.pallas import tpu as pltpu
from jax.experimental.pallas import tpu_sc as plsc

N = 10000
D = 32
F_IN = 128
HID = 256
NCLS = 16

NP = 10240
BLK = 512
NSC = 2
NTILE = 16
NW = NSC * NTILE


NBUF = 4


def _sc_gather(table, idx3, feat, dtype):
    _, n_chunks, chunk = idx3.shape
    per_w = n_chunks * chunk
    rows_total = NW * per_w
    n_iters = n_chunks // NBUF
    mesh = plsc.VectorSubcoreMesh(core_axis_name="c", subcore_axis_name="s")

    @functools.partial(
        pl.kernel,
        mesh=mesh,
        out_type=jax.ShapeDtypeStruct((rows_total, feat), dtype),
        scratch_types=[
            pltpu.VMEM((n_chunks, chunk), jnp.int32),
        ]
        + [pltpu.VMEM((chunk, feat), dtype) for _ in range(NBUF)]
        + [pltpu.SemaphoreType.DMA for _ in range(2 * NBUF)],
    )
    def gk(table_hbm, idx_hbm, out_hbm, idx_v, *rest):
        bufs = rest[:NBUF]
        gsems = rest[NBUF:2 * NBUF]
        osems = rest[2 * NBUF:]
        wid = lax.axis_index("s") * NSC + lax.axis_index("c")
        base = wid * per_w
        pltpu.sync_copy(idx_hbm.at[wid], idx_v)
        for b in range(NBUF):
            pltpu.async_copy(table_hbm.at[idx_v.at[b]], bufs[b], gsems[b])

        def body(k, carry):
            for b in range(NBUF):
                j = k * NBUF + b
                pltpu.make_async_copy(
                    table_hbm.at[idx_v.at[j]], bufs[b], gsems[b]).wait()
                out_slice = out_hbm.at[pl.ds(base + j * chunk, chunk)]
                pltpu.async_copy(bufs[b], out_slice, osems[b])

                @pl.when(k < n_iters - 1)
                def _():
                    pltpu.make_async_copy(bufs[b], out_slice, osems[b]).wait()
                    pltpu.async_copy(
                        table_hbm.at[idx_v.at[j + NBUF]], bufs[b], gsems[b])
            return carry

        lax.fori_loop(0, n_iters, body, 0)
        for b in range(NBUF):
            j = (n_iters - 1) * NBUF + b
            out_slice = out_hbm.at[pl.ds(base + j * chunk, chunk)]
            pltpu.make_async_copy(bufs[b], out_slice, osems[b]).wait()

    return gk(table, idx3)


def _conv1(m, xp, W_ihT, W_hhT, bias, W_self1, W_neigh1, b1, W_pool, b_pool):
    def body(m_ref, x_ref, wih_ref, whh_ref, b_ref, ws_ref, wn_ref, b1_ref,
             wp_ref, bp_ref, out_ref, q_ref, h_s, c_s):
        d = pl.program_id(1)

        @pl.when(d == 0)
        def _():
            h_s[...] = jnp.zeros_like(h_s)
            c_s[...] = jnp.zeros_like(c_s)

        xt = m_ref[0].astype(jnp.bfloat16)
        hb = h_s[...]
        pre_if = (jnp.dot(xt, wih_ref[:, 0:2 * F_IN],
                          preferred_element_type=jnp.float32)
                  + jnp.dot(hb, whh_ref[:, 0:2 * F_IN],
                            preferred_element_type=jnp.float32)
                  + b_ref[:, 0:2 * F_IN])
        pre_go = (jnp.dot(xt, wih_ref[:, 2 * F_IN:4 * F_IN],
                          preferred_element_type=jnp.float32)
                  + jnp.dot(hb, whh_ref[:, 2 * F_IN:4 * F_IN],
                            preferred_element_type=jnp.float32)
                  + b_ref[:, 2 * F_IN:4 * F_IN])
        def sigm(v):
            return 0.5 * jnp.tanh(v) + 0.5

        gi = sigm(pre_if[:, 0:F_IN])
        gf = sigm(pre_if[:, F_IN:2 * F_IN])
        gg = jnp.tanh(pre_go[:, 0:F_IN])
        go = sigm(pre_go[:, F_IN:2 * F_IN])
        c = gf * c_s[...] + gi * gg
        h = go * jnp.tanh(c)
        c_s[...] = c
        h_s[...] = h.astype(jnp.bfloat16)

        @pl.when(d == D - 1)
        def _():
            hh = jax.nn.relu(
                jnp.dot(x_ref[...], ws_ref[...], preferred_element_type=jnp.float32)
                + jnp.dot(h.astype(jnp.bfloat16), wn_ref[...],
                          preferred_element_type=jnp.float32)
                + b1_ref[...])
            out_ref[...] = hh
            q_ref[...] = jax.nn.relu(
                jnp.dot(hh.astype(jnp.bfloat16), wp_ref[...],
                        preferred_element_type=jnp.float32)
                + bp_ref[...])

    cn = m.shape[1]
    return pl.pallas_call(
        body,
        grid=(cn // BLK, D),
        in_specs=[
            pl.BlockSpec((1, BLK, F_IN), lambda i, d: (d, i, 0)),
            pl.BlockSpec((BLK, F_IN), lambda i, d: (i, 0)),
            pl.BlockSpec((F_IN, 4 * F_IN), lambda i, d: (0, 0)),
            pl.BlockSpec((F_IN, 4 * F_IN), lambda i, d: (0, 0)),
            pl.BlockSpec((1, 4 * F_IN), lambda i, d: (0, 0)),
            pl.BlockSpec((F_IN, HID), lambda i, d: (0, 0)),
            pl.BlockSpec((F_IN, HID), lambda i, d: (0, 0)),
            pl.BlockSpec((1, HID), lambda i, d: (0, 0)),
            pl.BlockSpec((HID, HID), lambda i, d: (0, 0)),
            pl.BlockSpec((1, HID), lambda i, d: (0, 0)),
        ],
        out_specs=[
            pl.BlockSpec((BLK, HID), lambda i, d: (i, 0)),
            pl.BlockSpec((BLK, HID), lambda i, d: (i, 0)),
        ],
        out_shape=[
            jax.ShapeDtypeStruct((cn, HID), jnp.float32),
            jax.ShapeDtypeStruct((cn, HID), jnp.float32),
        ],
        scratch_shapes=[
            pltpu.VMEM((BLK, F_IN), jnp.bfloat16),
            pltpu.VMEM((BLK, F_IN), jnp.float32),
        ],
    )(m, xp, W_ihT, W_hhT, bias, W_self1, W_neigh1, b1, W_pool, b_pool)


def _conv2(m2, h, W_self2, W_neigh2, b2):
    def body(m_ref, h_ref, ws_ref, wn_ref, b2_ref, out_ref, mx_s):
        d = pl.program_id(1)
        t = m_ref[0]
        prev = jnp.where(d == 0, jnp.zeros_like(t), mx_s[...])
        mx = jnp.maximum(t, prev)
        mx_s[...] = mx

        @pl.when(d == D - 1)
        def _():
            out_ref[...] = (
                jnp.dot(h_ref[...].astype(jnp.bfloat16), ws_ref[...],
                        preferred_element_type=jnp.float32)
                + jnp.dot(mx.astype(jnp.bfloat16), wn_ref[...],
                          preferred_element_type=jnp.float32)
                + b2_ref[...])

    cn = m2.shape[1]
    return pl.pallas_call(
        body,
        grid=(cn // BLK, D),
        in_specs=[
            pl.BlockSpec((1, BLK, HID), lambda i, d: (d, i, 0)),
            pl.BlockSpec((BLK, HID), lambda i, d: (i, 0)),
            pl.BlockSpec((HID, NCLS), lambda i, d: (0, 0)),
            pl.BlockSpec((HID, NCLS), lambda i, d: (0, 0)),
            pl.BlockSpec((1, NCLS), lambda i, d: (0, 0)),
        ],
        out_specs=pl.BlockSpec((BLK, NCLS), lambda i, d: (i, 0)),
        out_shape=jax.ShapeDtypeStruct((cn, NCLS), jnp.float32),
        scratch_shapes=[pltpu.VMEM((BLK, HID), jnp.float32)],
    )(m2, h, W_self2, W_neigh2, b2)


CHUNKS_A = (2560, 2560, 2560, 2560)
CHUNKS_B = (2560, 2560, 2560, 2560)


def kernel(x, nbr, W_ih, W_hh, b_ih, b_hh, W_self1, W_neigh1, b1,
           W_pool, b_pool, W_self2, W_neigh2, b2):
    bf = jnp.bfloat16
    nbr = nbr.astype(jnp.int32)
    xp = jnp.pad(x.astype(bf), ((0, NP - N), (0, 0)))
    idxt = jnp.pad(nbr.T, ((0, 0), (0, NP - N)))
    c1 = 16384 // F_IN
    c2 = 16384 // HID

    def chunk_idx(off, cn):
        return idxt[:, off:off + cn].reshape(-1)

    gate_scale = jnp.concatenate([
        jnp.full((2 * F_IN,), 0.5, jnp.float32),
        jnp.ones((F_IN,), jnp.float32),
        jnp.full((F_IN,), 0.5, jnp.float32),
    ])
    bias = ((b_ih + b_hh) * gate_scale).reshape(1, 4 * F_IN)
    wih = (W_ih.T * gate_scale[None, :]).astype(bf)
    whh = (W_hh.T * gate_scale[None, :]).astype(bf)
    ws1, wn1 = W_self1.astype(bf), W_neigh1.astype(bf)
    wp = W_pool.astype(bf)
    ws2, wn2 = W_self2.astype(bf), W_neigh2.astype(bf)

    hs, qs = [], []
    off = 0
    for cn in CHUNKS_A:
        per_w = (D * cn) // NW
        mk = _sc_gather(x, chunk_idx(off, cn).reshape(NW, per_w // c1, c1),
                        F_IN, jnp.float32).reshape(D, cn, F_IN)
        hk, qk = _conv1(mk, xp[off:off + cn], wih, whh, bias,
                        ws1, wn1, b1.reshape(1, HID), wp,
                        b_pool.reshape(1, HID))
        hs.append(hk)
        qs.append(qk)
        off += cn
    q = jnp.concatenate(qs, axis=0)
    h = jnp.concatenate(hs, axis=0)

    outs = []
    off = 0
    for cn in CHUNKS_B:
        per_w = (D * cn) // NW
        m2k = _sc_gather(q, chunk_idx(off, cn).reshape(NW, per_w // c2, c2),
                         HID, jnp.float32).reshape(D, cn, HID)
        outs.append(_conv2(m2k, h[off:off + cn], ws2, wn2,
                           b2.reshape(1, NCLS)))
        off += cn
    return jnp.concatenate(outs, axis=0)[:N]
